# baseline (speedup 1.0000x reference)
import functools

import jax
import jax.numpy as jnp
from jax import lax
from jax.experimental import pallas as pl
from jax.experimental.pallas import tpu as pltpu
from jax.experimental.pallas import tpu_sc as plsc

_B, _N, _C = 4, 8192, 1024
_NC, _NS, _L = 2, 16, 16
_NW = _NC * _NS
_ROWS_PER_W = _N // _NW
_R = 32
_RB = _ROWS_PER_W // _R
_CHUNK = _R * _C


def _sc_body(tok_hbm, emb_hbm, out_hbm, emb_v, tok_v):
    wid = lax.axis_index("s") * _NC + lax.axis_index("c")
    for rb in range(_RB):
        row0 = wid * _ROWS_PER_W + rb * _R
        pltpu.sync_copy(emb_hbm.at[pl.ds(row0 * _C, _CHUNK)], emb_v)
        for b in range(_B):
            off = (b * _N) * _C + row0 * _C
            pltpu.sync_copy(tok_hbm.at[pl.ds(off, _CHUNK)], tok_v)

            def _add(i, _):
                tok_v[pl.ds(i * _L, _L)] += emb_v[pl.ds(i * _L, _L)]
                return 0

            lax.fori_loop(0, _CHUNK // _L, _add, 0)
            pltpu.sync_copy(tok_v, out_hbm.at[pl.ds(off, _CHUNK)])


@functools.partial(jax.jit, static_argnames=())
def _sc_add(tok_flat, emb_flat):
    mesh = plsc.VectorSubcoreMesh(core_axis_name="c", subcore_axis_name="s")
    return pl.kernel(
        _sc_body,
        out_type=jax.ShapeDtypeStruct((_B * _N * _C,), jnp.float32),
        mesh=mesh,
        scratch_types=[
            pltpu.VMEM((_CHUNK,), jnp.float32),
            pltpu.VMEM((_CHUNK,), jnp.float32),
        ],
    )(tok_flat, emb_flat)


def kernel(tokens, emb):
    b, n, c = tokens.shape
    out = _sc_add(tokens.reshape(-1), emb.reshape(-1))
    return out.reshape(b, n, c)


# speedup vs baseline: 1.7092x; 1.7092x over previous
import jax
import jax.numpy as jnp
from jax import lax
from jax.experimental import pallas as pl
from jax.experimental.pallas import tpu as pltpu
from jax.experimental.pallas import tpu_sc as plsc

_B, _N, _C = 4, 8192, 1024
_NC, _NS, _L = 2, 16, 16
_NW = _NC * _NS
_ROWS_PER_W = _N // _NW
_R = 16
_RB = _ROWS_PER_W // _R
_CHUNK = _R * _C
_NBUF = 3
_UNROLL = 8
_STEPS = _RB * _B


def _sc_body(tok_hbm, emb_hbm, out_hbm, emb_v,
             tok0, tok1, tok2, isem0, isem1, isem2, osem0, osem1, osem2):
    tok_bufs = (tok0, tok1, tok2)
    in_sems = (isem0, isem1, isem2)
    out_sems = (osem0, osem1, osem2)
    wid = lax.axis_index("s") * _NC + lax.axis_index("c")
    row_base = wid * _ROWS_PER_W

    def tok_off(step):
        rb, b = step // _B, step % _B
        return (b * _N + row_base + rb * _R) * _C

    def start_in(step):
        p = step % _NBUF
        pltpu.async_copy(
            tok_hbm.at[pl.ds(tok_off(step), _CHUNK)], tok_bufs[p], in_sems[p])

    def wait_in(step):
        p = step % _NBUF
        pltpu.make_async_copy(
            tok_hbm.at[pl.ds(tok_off(step), _CHUNK)], tok_bufs[p],
            in_sems[p]).wait()

    def start_out(step):
        p = step % _NBUF
        pltpu.async_copy(
            tok_bufs[p], out_hbm.at[pl.ds(tok_off(step), _CHUNK)], out_sems[p])

    def wait_out(step):
        p = step % _NBUF
        pltpu.make_async_copy(
            tok_bufs[p], out_hbm.at[pl.ds(tok_off(step), _CHUNK)],
            out_sems[p]).wait()

    for s in range(_NBUF - 1):
        start_in(s)

    for s in range(_STEPS):
        p = s % _NBUF
        rb, b = s // _B, s % _B
        if b == 0:
            pltpu.sync_copy(
                emb_hbm.at[pl.ds((row_base + rb * _R) * _C, _CHUNK)], emb_v)
        wait_in(s)

        def _add(i, _):
            base = i * (_L * _UNROLL)
            for u in range(_UNROLL):
                tok_bufs[p][pl.ds(base + u * _L, _L)] += (
                    emb_v[pl.ds(base + u * _L, _L)])
            return 0

        lax.fori_loop(0, _CHUNK // (_L * _UNROLL), _add, 0)
        if s >= 1:
            wait_out(s - 1)
        if s + _NBUF - 1 < _STEPS:
            start_in(s + _NBUF - 1)
        start_out(s)

    wait_out(_STEPS - 1)


@jax.jit
def _sc_add(tok_flat, emb_flat):
    mesh = plsc.VectorSubcoreMesh(core_axis_name="c", subcore_axis_name="s")
    return pl.kernel(
        _sc_body,
        out_type=jax.ShapeDtypeStruct((_B * _N * _C,), jnp.float32),
        mesh=mesh,
        scratch_types=(
            [pltpu.VMEM((_CHUNK,), jnp.float32)]
            + [pltpu.VMEM((_CHUNK,), jnp.float32) for _ in range(_NBUF)]
            + [pltpu.SemaphoreType.DMA for _ in range(2 * _NBUF)]
        ),
    )(tok_flat, emb_flat)


def kernel(tokens, emb):
    b, n, c = tokens.shape
    out = _sc_add(tokens.reshape(-1), emb.reshape(-1))
    return out.reshape(b, n, c)


# speedup vs baseline: 1.7129x; 1.0021x over previous
import jax
import jax.numpy as jnp
from jax import lax
from jax.experimental import pallas as pl
from jax.experimental.pallas import tpu as pltpu
from jax.experimental.pallas import tpu_sc as plsc

_B, _N, _C = 4, 8192, 1024
_NC, _NS, _L = 2, 16, 16
_NW = _NC * _NS
_ROWS_PER_W = _N // _NW
_R = 16
_RB = _ROWS_PER_W // _R
_CHUNK = _R * _C
_NBUF = 3
_UNROLL = 8
_STEPS = _RB * _B


def _sc_body(tok_hbm, emb_hbm, out_hbm, emb_v,
             tok0, tok1, tok2, isem0, isem1, isem2, osem0, osem1, osem2):
    tok_bufs = (tok0, tok1, tok2)
    in_sems = (isem0, isem1, isem2)
    out_sems = (osem0, osem1, osem2)
    wid = lax.axis_index("s") * _NC + lax.axis_index("c")
    row_base = wid * _ROWS_PER_W

    def tok_off(step):
        rb, b = step // _B, step % _B
        return (b * _N + row_base + rb * _R) * _C

    def start_in(step):
        p = step % _NBUF
        pltpu.async_copy(
            tok_hbm.at[pl.ds(tok_off(step), _CHUNK)], tok_bufs[p], in_sems[p])

    def wait_in(step):
        p = step % _NBUF
        pltpu.make_async_copy(
            tok_hbm.at[pl.ds(tok_off(step), _CHUNK)], tok_bufs[p],
            in_sems[p]).wait()

    def start_out(step):
        p = step % _NBUF
        pltpu.async_copy(
            tok_bufs[p], out_hbm.at[pl.ds(tok_off(step), _CHUNK)], out_sems[p])

    def wait_out(step):
        p = step % _NBUF
        pltpu.make_async_copy(
            tok_bufs[p], out_hbm.at[pl.ds(tok_off(step), _CHUNK)],
            out_sems[p]).wait()

    for s in range(_NBUF - 1):
        start_in(s)

    for s in range(_STEPS):
        p = s % _NBUF
        rb, b = s // _B, s % _B
        if b == 0:
            pltpu.sync_copy(
                emb_hbm.at[pl.ds((row_base + rb * _R) * _C, _CHUNK)], emb_v)
        wait_in(s)

        @plsc.parallel_loop(0, _CHUNK, _L, unroll=_UNROLL)
        def _add(i):
            tok_bufs[p][pl.ds(i, _L)] += emb_v[pl.ds(i, _L)]
        if s >= 1:
            wait_out(s - 1)
        if s + _NBUF - 1 < _STEPS:
            start_in(s + _NBUF - 1)
        start_out(s)

    wait_out(_STEPS - 1)


@jax.jit
def _sc_add(tok_flat, emb_flat):
    mesh = plsc.VectorSubcoreMesh(core_axis_name="c", subcore_axis_name="s")
    return pl.kernel(
        _sc_body,
        out_type=jax.ShapeDtypeStruct((_B * _N * _C,), jnp.float32),
        mesh=mesh,
        scratch_types=(
            [pltpu.VMEM((_CHUNK,), jnp.float32)]
            + [pltpu.VMEM((_CHUNK,), jnp.float32) for _ in range(_NBUF)]
            + [pltpu.SemaphoreType.DMA for _ in range(2 * _NBUF)]
        ),
    )(tok_flat, emb_flat)


def kernel(tokens, emb):
    b, n, c = tokens.shape
    out = _sc_add(tokens.reshape(-1), emb.reshape(-1))
    return out.reshape(b, n, c)


# speedup vs baseline: 4.6780x; 2.7311x over previous
import jax
import jax.numpy as jnp
from jax import lax
from jax.experimental import pallas as pl
from jax.experimental.pallas import tpu as pltpu
from jax.experimental.pallas import tpu_sc as plsc

_B, _N, _C = 4, 8192, 1024
_NC, _NS, _L = 2, 16, 16
_NW = _NC * _NS
_ROWS_PER_W = _N // _NW
_R = 16
_RB = _ROWS_PER_W // _R
_NBUF = 3
_UNROLL = 8
_STEPS = _RB * _B


def _sc_body(tok_hbm, emb_hbm, out_hbm, emb_v,
             tok0, tok1, tok2, isem0, isem1, isem2, osem0, osem1, osem2):
    tok_bufs = (tok0, tok1, tok2)
    in_sems = (isem0, isem1, isem2)
    out_sems = (osem0, osem1, osem2)
    wid = lax.axis_index("s") * _NC + lax.axis_index("c")
    row_base = wid * _ROWS_PER_W

    def rows(step):
        rb, b = step // _B, step % _B
        return b, row_base + rb * _R

    def start_in(step):
        p = step % _NBUF
        b, r0 = rows(step)
        pltpu.async_copy(
            tok_hbm.at[b, pl.ds(r0, _R), :], tok_bufs[p], in_sems[p])

    def wait_in(step):
        p = step % _NBUF
        b, r0 = rows(step)
        pltpu.make_async_copy(
            tok_hbm.at[b, pl.ds(r0, _R), :], tok_bufs[p], in_sems[p]).wait()

    def start_out(step):
        p = step % _NBUF
        b, r0 = rows(step)
        pltpu.async_copy(
            tok_bufs[p], out_hbm.at[b, pl.ds(r0, _R), :], out_sems[p])

    def wait_out(step):
        p = step % _NBUF
        b, r0 = rows(step)
        pltpu.make_async_copy(
            tok_bufs[p], out_hbm.at[b, pl.ds(r0, _R), :], out_sems[p]).wait()

    for s in range(_NBUF - 1):
        start_in(s)

    for s in range(_STEPS):
        p = s % _NBUF
        rb, b = s // _B, s % _B
        if b == 0:
            pltpu.sync_copy(
                emb_hbm.at[pl.ds(row_base + rb * _R, _R), :], emb_v)
        wait_in(s)

        @plsc.parallel_loop(0, _R * _C, _L, unroll=_UNROLL)
        def _add(i):
            r = i >> 10
            c = pl.multiple_of(i & (_C - 1), _L)
            tok_bufs[p][r, pl.ds(c, _L)] += emb_v[r, pl.ds(c, _L)]

        if s >= 1:
            wait_out(s - 1)
        if s + _NBUF - 1 < _STEPS:
            start_in(s + _NBUF - 1)
        start_out(s)

    wait_out(_STEPS - 1)


@jax.jit
def _sc_add(tokens, emb):
    mesh = plsc.VectorSubcoreMesh(core_axis_name="c", subcore_axis_name="s")
    return pl.kernel(
        _sc_body,
        out_type=jax.ShapeDtypeStruct((_B, _N, _C), jnp.float32),
        mesh=mesh,
        scratch_types=(
            [pltpu.VMEM((_R, _C), jnp.float32)]
            + [pltpu.VMEM((_R, _C), jnp.float32) for _ in range(_NBUF)]
            + [pltpu.SemaphoreType.DMA for _ in range(2 * _NBUF)]
        ),
    )(tokens, emb)


def kernel(tokens, emb):
    return _sc_add(tokens, emb)


# speedup vs baseline: 4.8367x; 1.0339x over previous
import jax
import jax.numpy as jnp
from jax import lax
from jax.experimental import pallas as pl
from jax.experimental.pallas import tpu as pltpu
from jax.experimental.pallas import tpu_sc as plsc

_B, _N, _C = 4, 8192, 1024
_NC, _NS, _L = 2, 16, 16
_NW = _NC * _NS
_N_SC = 3072
_ROWS_PER_W = _N_SC // _NW
_R = 16
_RB = _ROWS_PER_W // _R
_NBUF = 3
_UNROLL = 8
_STEPS = _RB * _B
_BN = 1024


def _sc_body(tok_hbm, emb_hbm, out_hbm, emb_v,
             tok0, tok1, tok2, isem0, isem1, isem2, osem0, osem1, osem2):
    tok_bufs = (tok0, tok1, tok2)
    in_sems = (isem0, isem1, isem2)
    out_sems = (osem0, osem1, osem2)
    wid = lax.axis_index("s") * _NC + lax.axis_index("c")
    row_base = wid * _ROWS_PER_W

    def rows(step):
        rb, b = step // _B, step % _B
        return b, row_base + rb * _R

    def start_in(step):
        p = step % _NBUF
        b, r0 = rows(step)
        pltpu.async_copy(
            tok_hbm.at[b, pl.ds(r0, _R), :], tok_bufs[p], in_sems[p])

    def wait_in(step):
        p = step % _NBUF
        b, r0 = rows(step)
        pltpu.make_async_copy(
            tok_hbm.at[b, pl.ds(r0, _R), :], tok_bufs[p], in_sems[p]).wait()

    def start_out(step):
        p = step % _NBUF
        b, r0 = rows(step)
        pltpu.async_copy(
            tok_bufs[p], out_hbm.at[b, pl.ds(r0, _R), :], out_sems[p])

    def wait_out(step):
        p = step % _NBUF
        b, r0 = rows(step)
        pltpu.make_async_copy(
            tok_bufs[p], out_hbm.at[b, pl.ds(r0, _R), :], out_sems[p]).wait()

    for s in range(_NBUF - 1):
        start_in(s)

    for s in range(_STEPS):
        p = s % _NBUF
        rb, b = s // _B, s % _B
        if b == 0:
            pltpu.sync_copy(
                emb_hbm.at[pl.ds(row_base + rb * _R, _R), :], emb_v)
        wait_in(s)

        @plsc.parallel_loop(0, _R * _C, _L, unroll=_UNROLL)
        def _add(i):
            r = i >> 10
            c = pl.multiple_of(i & (_C - 1), _L)
            tok_bufs[p][r, pl.ds(c, _L)] += emb_v[r, pl.ds(c, _L)]

        if s >= 1:
            wait_out(s - 1)
        if s + _NBUF - 1 < _STEPS:
            start_in(s + _NBUF - 1)
        start_out(s)

    wait_out(_STEPS - 1)


def _sc_add(tokens, emb):
    mesh = plsc.VectorSubcoreMesh(core_axis_name="c", subcore_axis_name="s")
    return pl.kernel(
        _sc_body,
        out_type=jax.ShapeDtypeStruct((_B, _N_SC, _C), jnp.float32),
        mesh=mesh,
        scratch_types=(
            [pltpu.VMEM((_R, _C), jnp.float32)]
            + [pltpu.VMEM((_R, _C), jnp.float32) for _ in range(_NBUF)]
            + [pltpu.SemaphoreType.DMA for _ in range(2 * _NBUF)]
        ),
    )(tokens, emb)


def _tc_add_body(tok_ref, emb_ref, out_ref):
    out_ref[...] = tok_ref[...] + emb_ref[...]


def _tc_add(tokens, emb):
    base = _N_SC // _BN
    grid = ((_N - _N_SC) // _BN, _B)
    return pl.pallas_call(
        _tc_add_body,
        grid=grid,
        in_specs=[
            pl.BlockSpec((1, _BN, _C), lambda i, j: (j, i + base, 0)),
            pl.BlockSpec((_BN, _C), lambda i, j: (i + base, 0)),
        ],
        out_specs=pl.BlockSpec((1, _BN, _C), lambda i, j: (j, i + base, 0)),
        out_shape=jax.ShapeDtypeStruct((_B, _N, _C), jnp.float32),
    )(tokens, emb)


def kernel(tokens, emb):
    sc_part = _sc_add(tokens, emb)
    tc_full = _tc_add(tokens, emb)
    return lax.dynamic_update_slice(tc_full, sc_part, (0, 0, 0))
